# Initial kernel scaffold; baseline (speedup 1.0000x reference)
import functools

import jax
import jax.numpy as jnp
from jax import lax
from jax.experimental import pallas as pl
from jax.experimental.pallas import tpu as pltpu
from jax.experimental.pallas import tpu_sc as plsc

N = 10000
E = 320000
D = 128
G = 64
L = 3

C = 80
NW = 32
CPW = E // C // NW
RPT = N // 16

BLK = 1000
NBLK = N // BLK


def _sc_aggregate(h, src2d, dst2d, ew2d, zeros):
    mesh = plsc.VectorSubcoreMesh(core_axis_name="c", subcore_axis_name="s")

    @functools.partial(
        pl.kernel,
        out_type=jax.ShapeDtypeStruct((2, N, D), jnp.float32),
        mesh=mesh,
        scratch_types=[
            pltpu.VMEM((CPW, C), jnp.int32),
            pltpu.VMEM((CPW, C), jnp.int32),
            pltpu.VMEM((CPW, C), jnp.float32),
            pltpu.VMEM((C, D), jnp.float32),
            pltpu.VMEM_SHARED((N, D), jnp.float32),
            pltpu.SemaphoreType.DMA,
        ],
    )
    def k(h_hbm, src_hbm, dst_hbm, ew_hbm, z_hbm, out_hbm,
          src_v, dst_v, ew_v, rows_v, acc_sh, sem):
        c = lax.axis_index("c")
        s = lax.axis_index("s")
        w = s * 2 + c

        pltpu.sync_copy(z_hbm.at[pl.ds(s * RPT, RPT)],
                        acc_sh.at[pl.ds(s * RPT, RPT)])
        pltpu.sync_copy(src_hbm.at[pl.ds(w * CPW, CPW)], src_v)
        pltpu.sync_copy(dst_hbm.at[pl.ds(w * CPW, CPW)], dst_v)
        pltpu.sync_copy(ew_hbm.at[pl.ds(w * CPW, CPW)], ew_v)
        plsc.subcore_barrier()

        iota = lax.iota(jnp.int32, 16)

        def chunk_body(kk, carry):
            pltpu.async_copy(h_hbm.at[src_v.at[kk]], rows_v, sem).wait()
            kvec = jnp.full((16,), kk, jnp.int32)

            def edge_body(e, carry2):
                evec = jnp.full((16,), e, jnp.int32)
                wv = plsc.load_gather(ew_v, [kvec, evec])
                for j in range(D // 16):
                    cols = iota + (j * 16)
                    v = plsc.load_gather(rows_v, [evec, cols])
                    plsc.store_scatter(rows_v, [evec, cols], v * wv)
                return carry2

            lax.fori_loop(0, C, edge_body, 0)
            pltpu.sync_copy(rows_v, acc_sh.at[dst_v.at[kk]], add=True)
            return carry

        lax.fori_loop(0, CPW, chunk_body, 0)
        plsc.subcore_barrier()
        pltpu.sync_copy(acc_sh.at[pl.ds(s * RPT, RPT)],
                        out_hbm.at[c].at[pl.ds(s * RPT, RPT)])

    return k(h, src2d, dst2d, ew2d, zeros)


def _mlp_body(h_ref, p0_ref, p1_ref, w1_ref, b1_ref, w2_ref, b2_ref,
              u_ref, s_ref):
    i = pl.program_id(0)
    z = h_ref[...] + p0_ref[...] + p1_ref[...]
    a = jnp.maximum(
        jnp.dot(z, w1_ref[...], preferred_element_type=jnp.float32)
        + b1_ref[...], 0.0)
    u = jnp.maximum(
        jnp.dot(a, w2_ref[...], preferred_element_type=jnp.float32)
        + b2_ref[...], 0.0)
    u_ref[...] = u
    su = jnp.sum(u, axis=0, keepdims=True)
    sq = jnp.sum(u * u, axis=0, keepdims=True)
    st = jnp.concatenate([su, sq], axis=0)

    @pl.when(i == 0)
    def _():
        s_ref[...] = jnp.zeros_like(s_ref)

    s_ref[...] += st


def _mlp(h, p0, p1, W1, b1, W2, b2):
    return pl.pallas_call(
        _mlp_body,
        grid=(NBLK,),
        in_specs=[
            pl.BlockSpec((BLK, D), lambda i: (i, 0)),
            pl.BlockSpec((BLK, D), lambda i: (i, 0)),
            pl.BlockSpec((BLK, D), lambda i: (i, 0)),
            pl.BlockSpec((D, D), lambda i: (0, 0)),
            pl.BlockSpec((1, D), lambda i: (0, 0)),
            pl.BlockSpec((D, D), lambda i: (0, 0)),
            pl.BlockSpec((1, D), lambda i: (0, 0)),
        ],
        out_specs=[
            pl.BlockSpec((BLK, D), lambda i: (i, 0)),
            pl.BlockSpec((2, D), lambda i: (0, 0)),
        ],
        out_shape=[
            jax.ShapeDtypeStruct((N, D), jnp.float32),
            jax.ShapeDtypeStruct((2, D), jnp.float32),
        ],
    )(h, p0, p1, W1, b1.reshape(1, D), W2, b2.reshape(1, D))


def _bn_pool_body(u_ref, s_ref, g_ref, be_ref, bt_ref, z_ref, pool_ref):
    i = pl.program_id(0)
    mean = s_ref[0:1, :] * (1.0 / N)
    var = s_ref[1:2, :] * (1.0 / N) - mean * mean
    scale = lax.rsqrt(var + 1e-5) * g_ref[...]
    z = (u_ref[...] - mean) * scale + be_ref[...]
    z_ref[...] = z
    b = bt_ref[0, 0, :]
    gi = lax.broadcasted_iota(jnp.int32, (BLK, G), 1)
    onehot = (b[:, None] == gi).astype(jnp.float32)
    contrib = lax.dot_general(onehot, z, (((0,), (0,)), ((), ())),
                              preferred_element_type=jnp.float32)

    @pl.when(i == 0)
    def _():
        pool_ref[...] = jnp.zeros_like(pool_ref)

    pool_ref[...] += contrib


def _bn_pool(u, s, gamma, beta, batch3d):
    return pl.pallas_call(
        _bn_pool_body,
        grid=(NBLK,),
        in_specs=[
            pl.BlockSpec((BLK, D), lambda i: (i, 0)),
            pl.BlockSpec((2, D), lambda i: (0, 0)),
            pl.BlockSpec((1, D), lambda i: (0, 0)),
            pl.BlockSpec((1, D), lambda i: (0, 0)),
            pl.BlockSpec((1, 1, BLK), lambda i: (i, 0, 0)),
        ],
        out_specs=[
            pl.BlockSpec((BLK, D), lambda i: (i, 0)),
            pl.BlockSpec((G, D), lambda i: (0, 0)),
        ],
        out_shape=[
            jax.ShapeDtypeStruct((N, D), jnp.float32),
            jax.ShapeDtypeStruct((G, D), jnp.float32),
        ],
    )(u, s, gamma.reshape(1, D), beta.reshape(1, D), batch3d)


def kernel(x, edge_index, batch, edge_weight,
           W1_0, b1_0, W2_0, b2_0, gamma_0, beta_0,
           W1_1, b1_1, W2_1, b2_1, gamma_1, beta_1,
           W1_2, b1_2, W2_2, b2_2, gamma_2, beta_2):
    params = [
        (W1_0, b1_0, W2_0, b2_0, gamma_0, beta_0),
        (W1_1, b1_1, W2_1, b2_1, gamma_1, beta_1),
        (W1_2, b1_2, W2_2, b2_2, gamma_2, beta_2),
    ]
    src2d = edge_index[0].reshape(E // C, C)
    dst2d = edge_index[1].reshape(E // C, C)
    ew2d = edge_weight.reshape(E // C, C)
    zeros = jnp.zeros((N, D), jnp.float32)
    batch3d = batch.reshape(NBLK, 1, BLK)

    h = x
    zs, pools = [], []
    for (W1, b1, W2, b2, ga, be) in params:
        parts = _sc_aggregate(h, src2d, dst2d, ew2d, zeros)
        u, s = _mlp(h, parts[0], parts[1], W1, b1, W2, b2)
        z, pool = _bn_pool(u, s, ga, be, batch3d)
        zs.append(z)
        pools.append(pool)
        h = z
    x_g = jnp.concatenate(pools, axis=1)
    x_all = jnp.concatenate(zs, axis=1)
    return (x_g, x_all)


# speedup vs baseline: 5.5304x; 5.5304x over previous
import functools

import jax
import jax.numpy as jnp
from jax import lax
from jax.experimental import pallas as pl
from jax.experimental.pallas import tpu as pltpu
from jax.experimental.pallas import tpu_sc as plsc

N = 10000
E = 320000
D = 128
G = 64
L = 3

C = 80
NW = 32
CPW = E // C // NW
RPT = 624

BLK = 1000
NBLK = N // BLK


def _sc_aggregate(h, src2d, dst2d, ew2d, zeros):
    mesh = plsc.VectorSubcoreMesh(core_axis_name="c", subcore_axis_name="s")

    @functools.partial(
        pl.kernel,
        out_type=jax.ShapeDtypeStruct((2, N, D), jnp.float32),
        mesh=mesh,
        scratch_types=[
            pltpu.VMEM((CPW * C,), jnp.int32),
            pltpu.VMEM((CPW, C), jnp.int32),
            pltpu.VMEM((CPW * C,), jnp.float32),
            pltpu.VMEM((C, D), jnp.float32),
            pltpu.VMEM_SHARED((N, D), jnp.float32),
            pltpu.SemaphoreType.DMA,
        ],
    )
    def k(h_hbm, src_hbm, dst_hbm, ew_hbm, z_hbm, out_hbm,
          src_v, dst_v, ew_v, rows_v, acc_sh, sem):
        c = lax.axis_index("c")
        s = lax.axis_index("s")
        w = s * 2 + c

        pltpu.sync_copy(z_hbm.at[pl.ds(s * RPT, RPT)],
                        acc_sh.at[pl.ds(s * RPT, RPT)])

        @pl.when(s == 15)
        def _():
            pltpu.sync_copy(z_hbm.at[pl.ds(16 * RPT, N - 16 * RPT)],
                            acc_sh.at[pl.ds(16 * RPT, N - 16 * RPT)])

        pltpu.sync_copy(src_hbm.at[w], src_v)
        pltpu.sync_copy(dst_hbm.at[w], dst_v)
        pltpu.sync_copy(ew_hbm.at[w], ew_v)
        plsc.subcore_barrier()

        def chunk_body(kk, carry):
            pltpu.async_copy(h_hbm.at[src_v.at[pl.ds(kk * C, C)]],
                             rows_v, sem).wait()

            def group_body(g, carry2):
                wrow = ew_v[pl.ds(kk * C + g * 16, 16)]
                for l in range(16):
                    e = g * 16 + l
                    wv = jnp.full((16,), wrow[l])
                    for j in range(D // 16):
                        rows_v[e, pl.ds(j * 16, 16)] = (
                            rows_v[e, pl.ds(j * 16, 16)] * wv)
                return carry2

            lax.fori_loop(0, C // 16, group_body, 0)
            pltpu.sync_copy(rows_v, acc_sh.at[dst_v.at[kk]], add=True)
            return carry

        lax.fori_loop(0, CPW, chunk_body, 0)
        plsc.subcore_barrier()
        pltpu.sync_copy(acc_sh.at[pl.ds(s * RPT, RPT)],
                        out_hbm.at[c].at[pl.ds(s * RPT, RPT)])

        @pl.when(s == 15)
        def _():
            pltpu.sync_copy(acc_sh.at[pl.ds(16 * RPT, N - 16 * RPT)],
                            out_hbm.at[c].at[pl.ds(16 * RPT, N - 16 * RPT)])

    return k(h, src2d, dst2d, ew2d, zeros)


def _mlp_body(h_ref, p0_ref, p1_ref, w1_ref, b1_ref, w2_ref, b2_ref,
              u_ref, s_ref):
    i = pl.program_id(0)
    z = h_ref[...] + p0_ref[...] + p1_ref[...]
    a = jnp.maximum(
        jnp.dot(z, w1_ref[...], preferred_element_type=jnp.float32)
        + b1_ref[...], 0.0)
    u = jnp.maximum(
        jnp.dot(a, w2_ref[...], preferred_element_type=jnp.float32)
        + b2_ref[...], 0.0)
    u_ref[...] = u
    su = jnp.sum(u, axis=0, keepdims=True)
    sq = jnp.sum(u * u, axis=0, keepdims=True)
    st = jnp.concatenate([su, sq], axis=0)

    @pl.when(i == 0)
    def _():
        s_ref[...] = jnp.zeros_like(s_ref)

    s_ref[...] += st


def _mlp(h, p0, p1, W1, b1, W2, b2):
    return pl.pallas_call(
        _mlp_body,
        grid=(NBLK,),
        in_specs=[
            pl.BlockSpec((BLK, D), lambda i: (i, 0)),
            pl.BlockSpec((BLK, D), lambda i: (i, 0)),
            pl.BlockSpec((BLK, D), lambda i: (i, 0)),
            pl.BlockSpec((D, D), lambda i: (0, 0)),
            pl.BlockSpec((1, D), lambda i: (0, 0)),
            pl.BlockSpec((D, D), lambda i: (0, 0)),
            pl.BlockSpec((1, D), lambda i: (0, 0)),
        ],
        out_specs=[
            pl.BlockSpec((BLK, D), lambda i: (i, 0)),
            pl.BlockSpec((2, D), lambda i: (0, 0)),
        ],
        out_shape=[
            jax.ShapeDtypeStruct((N, D), jnp.float32),
            jax.ShapeDtypeStruct((2, D), jnp.float32),
        ],
    )(h, p0, p1, W1, b1.reshape(1, D), W2, b2.reshape(1, D))


def _bn_pool_body(u_ref, s_ref, g_ref, be_ref, bt_ref, z_ref, pool_ref):
    i = pl.program_id(0)
    mean = s_ref[0:1, :] * (1.0 / N)
    var = s_ref[1:2, :] * (1.0 / N) - mean * mean
    scale = lax.rsqrt(var + 1e-5) * g_ref[...]
    z = (u_ref[...] - mean) * scale + be_ref[...]
    z_ref[...] = z
    b = bt_ref[0, 0, :]
    gi = lax.broadcasted_iota(jnp.int32, (BLK, G), 1)
    onehot = (b[:, None] == gi).astype(jnp.bfloat16)
    zhi = z.astype(jnp.bfloat16)
    zlo = (z - zhi.astype(jnp.float32)).astype(jnp.bfloat16)
    dn = (((0,), (0,)), ((), ()))
    contrib = (
        lax.dot_general(onehot, zhi, dn, preferred_element_type=jnp.float32)
        + lax.dot_general(onehot, zlo, dn, preferred_element_type=jnp.float32))

    @pl.when(i == 0)
    def _():
        pool_ref[...] = jnp.zeros_like(pool_ref)

    pool_ref[...] += contrib


def _bn_pool(u, s, gamma, beta, batch3d):
    return pl.pallas_call(
        _bn_pool_body,
        grid=(NBLK,),
        in_specs=[
            pl.BlockSpec((BLK, D), lambda i: (i, 0)),
            pl.BlockSpec((2, D), lambda i: (0, 0)),
            pl.BlockSpec((1, D), lambda i: (0, 0)),
            pl.BlockSpec((1, D), lambda i: (0, 0)),
            pl.BlockSpec((1, 1, BLK), lambda i: (i, 0, 0)),
        ],
        out_specs=[
            pl.BlockSpec((BLK, D), lambda i: (i, 0)),
            pl.BlockSpec((G, D), lambda i: (0, 0)),
        ],
        out_shape=[
            jax.ShapeDtypeStruct((N, D), jnp.float32),
            jax.ShapeDtypeStruct((G, D), jnp.float32),
        ],
    )(u, s, gamma.reshape(1, D), beta.reshape(1, D), batch3d)


def kernel(x, edge_index, batch, edge_weight,
           W1_0, b1_0, W2_0, b2_0, gamma_0, beta_0,
           W1_1, b1_1, W2_1, b2_1, gamma_1, beta_1,
           W1_2, b1_2, W2_2, b2_2, gamma_2, beta_2):
    params = [
        (W1_0, b1_0, W2_0, b2_0, gamma_0, beta_0),
        (W1_1, b1_1, W2_1, b2_1, gamma_1, beta_1),
        (W1_2, b1_2, W2_2, b2_2, gamma_2, beta_2),
    ]
    src2d = edge_index[0].reshape(NW, CPW * C)
    dst2d = edge_index[1].reshape(NW, CPW, C)
    ew2d = edge_weight.reshape(NW, CPW * C)
    zeros = jnp.zeros((N, D), jnp.float32)
    batch3d = batch.reshape(NBLK, 1, BLK)

    h = x
    zs, pools = [], []
    for (W1, b1, W2, b2, ga, be) in params:
        parts = _sc_aggregate(h, src2d, dst2d, ew2d, zeros)
        u, s = _mlp(h, parts[0], parts[1], W1, b1, W2, b2)
        z, pool = _bn_pool(u, s, ga, be, batch3d)
        zs.append(z)
        pools.append(pool)
        h = z
    x_g = jnp.concatenate(pools, axis=1)
    x_all = jnp.concatenate(zs, axis=1)
    return (x_g, x_all)
